# baseline (speedup 1.0000x reference)
import functools

import jax
import jax.numpy as jnp
from jax import lax
from jax.experimental import pallas as pl
from jax.experimental.pallas import tpu as pltpu
from jax.experimental.pallas import tpu_sc as plsc

T = 2048
H = 1024
F = 4096
E = 8
K = 2
P = T * K
R = 128
NB = P // R + E
P_PAD = NB * R
FT = 1024
NF = F // FT

NC = 2
NS = 16
NW = NC * NS


def _router_body(x_ref, w_ref, logits_ref, i1_ref, i2_ref, w1_ref, w2_ref):
    logits = jnp.dot(x_ref[...], w_ref[...], preferred_element_type=jnp.float32)
    logits_ref[...] = logits
    probs = jax.nn.softmax(logits, axis=-1)
    eidx = lax.broadcasted_iota(jnp.int32, (T, E), 1)
    m1 = jnp.max(probs, axis=1, keepdims=True)
    i1 = jnp.min(jnp.where(probs == m1, eidx, E), axis=1, keepdims=True)
    masked = jnp.where(eidx == i1, -1.0, probs)
    m2 = jnp.max(masked, axis=1, keepdims=True)
    i2 = jnp.min(jnp.where(masked == m2, eidx, E), axis=1, keepdims=True)
    s = m1 + m2
    i1_ref[...] = i1
    i2_ref[...] = i2
    w1_ref[...] = m1 / s
    w2_ref[...] = m2 / s


def _router(flat, router_w):
    return pl.pallas_call(
        _router_body,
        out_shape=(
            jax.ShapeDtypeStruct((T, E), jnp.float32),
            jax.ShapeDtypeStruct((T, 1), jnp.int32),
            jax.ShapeDtypeStruct((T, 1), jnp.int32),
            jax.ShapeDtypeStruct((T, 1), jnp.float32),
            jax.ShapeDtypeStruct((T, 1), jnp.float32),
        ),
    )(flat, router_w)


_DISP_ROWS = P_PAD // NW
_DISP_CHUNK = 40
_DISP_ITERS = _DISP_ROWS // _DISP_CHUNK


@functools.cache
def _sc_mesh():
    return plsc.VectorSubcoreMesh(core_axis_name="c", subcore_axis_name="s")


@functools.cache
def _build_dispatch():
    @functools.partial(
        pl.kernel,
        mesh=_sc_mesh(),
        out_type=jax.ShapeDtypeStruct((P_PAD, H), jnp.float32),
        scratch_types=[
            pltpu.VMEM((_DISP_ROWS,), jnp.int32),
            pltpu.VMEM((_DISP_CHUNK, H), jnp.float32),
            pltpu.VMEM((_DISP_CHUNK, H), jnp.float32),
            pltpu.SemaphoreType.DMA,
            pltpu.SemaphoreType.DMA,
            pltpu.SemaphoreType.DMA,
            pltpu.SemaphoreType.DMA,
        ],
    )
    def disp(x_hbm, src_hbm, xs_hbm, idx_v, rows0, rows1, g0, g1, s0, s1):
        wid = lax.axis_index("s") * NC + lax.axis_index("c")
        base = wid * _DISP_ROWS
        pltpu.sync_copy(src_hbm.at[pl.ds(base, _DISP_ROWS)], idx_v)
        bufs = (rows0, rows1)
        gsem = (g0, g1)
        ssem = (s0, s1)

        def fire(c):
            return pltpu.async_copy(
                x_hbm.at[idx_v.at[pl.ds(c * _DISP_CHUNK, _DISP_CHUNK)]],
                bufs[c & 1], gsem[c & 1])

        gathers = [None] * _DISP_ITERS
        stores = [None, None]
        gathers[0] = fire(0)
        for c in range(_DISP_ITERS):
            if c + 1 < _DISP_ITERS:
                b1 = (c + 1) & 1
                if stores[b1] is not None:
                    stores[b1].wait()
                gathers[c + 1] = fire(c + 1)
            gathers[c].wait()
            stores[c & 1] = pltpu.async_copy(
                bufs[c & 1],
                xs_hbm.at[pl.ds(base + c * _DISP_CHUNK, _DISP_CHUNK)],
                ssem[c & 1])
        stores[0].wait()
        stores[1].wait()

    return disp


def _dispatch(flat, tok_src):
    return _build_dispatch()(flat, tok_src)


def _gmm_body(eid_ref, vld_ref, xs_ref, gw_ref, uw_ref, dw_ref, wr_ref, out_ref):
    j = pl.program_id(0)
    i = pl.program_id(1)

    @pl.when(vld_ref[i] == 1)
    def _():
        dims = (((1,), (0,)), ((), ()))
        xb = xs_ref[...]
        g = lax.dot_general(xb, gw_ref[0], dims,
                            preferred_element_type=jnp.float32)
        u = lax.dot_general(xb, uw_ref[0], dims,
                            preferred_element_type=jnp.float32)
        hmid = g / (1.0 + jnp.exp(-g)) * u
        partial = lax.dot_general(hmid, dw_ref[0], dims,
                                  preferred_element_type=jnp.float32)
        rows = pl.ds(i * R, R)

        @pl.when(j == 0)
        def _():
            out_ref[rows, :] = partial

        @pl.when(j > 0)
        def _():
            out_ref[rows, :] = out_ref[rows, :] + partial

        @pl.when(j == NF - 1)
        def _():
            out_ref[rows, :] = out_ref[rows, :] * wr_ref[...]


def _gmm(eid, valid, xs, gate_w, up_w, down_w, wrow):
    grid_spec = pltpu.PrefetchScalarGridSpec(
        num_scalar_prefetch=2,
        grid=(NF, NB),
        in_specs=[
            pl.BlockSpec((R, H), lambda j, i, eid, vld: (i, 0)),
            pl.BlockSpec((1, H, FT), lambda j, i, eid, vld: (eid[i], 0, j)),
            pl.BlockSpec((1, H, FT), lambda j, i, eid, vld: (eid[i], 0, j)),
            pl.BlockSpec((1, FT, H), lambda j, i, eid, vld: (eid[i], j, 0)),
            pl.BlockSpec((R, 1), lambda j, i, eid, vld: (i, 0)),
        ],
        out_specs=pl.BlockSpec((P_PAD, H), lambda j, i, eid, vld: (0, 0)),
    )
    return pl.pallas_call(
        _gmm_body,
        grid_spec=grid_spec,
        out_shape=jax.ShapeDtypeStruct((P_PAD, H), jnp.float32),
        compiler_params=pltpu.CompilerParams(
            dimension_semantics=("arbitrary", "arbitrary"),
        ),
    )(eid, valid, xs, gate_w, up_w, down_w, wrow)


_COMB_ROWS = T // NW
_COMB_CHUNK = 32
_COMB_ITERS = _COMB_ROWS // _COMB_CHUNK
_VECS = _COMB_CHUNK * H // 16


@functools.cache
def _build_combine():
    @functools.partial(
        pl.kernel,
        mesh=_sc_mesh(),
        out_type=jax.ShapeDtypeStruct((T, H), jnp.float32),
        scratch_types=[
            pltpu.VMEM((_COMB_ROWS,), jnp.int32),
            pltpu.VMEM((_COMB_ROWS,), jnp.int32),
            pltpu.VMEM((_COMB_CHUNK, H), jnp.float32),
            pltpu.VMEM((_COMB_CHUNK, H), jnp.float32),
            pltpu.VMEM((_COMB_CHUNK, H), jnp.float32),
            pltpu.SemaphoreType.DMA,
            pltpu.SemaphoreType.DMA,
            pltpu.SemaphoreType.DMA,
            pltpu.SemaphoreType.DMA,
        ],
    )
    def comb(ys_hbm, p0_hbm, p1_hbm, out_hbm,
             i0_v, i1_v, a0_v, a1_v, b_v, s0, s1, t0, t1):
        wid = lax.axis_index("s") * NC + lax.axis_index("c")
        base = wid * _COMB_ROWS
        pltpu.sync_copy(p0_hbm.at[pl.ds(base, _COMB_ROWS)], i0_v)
        pltpu.sync_copy(p1_hbm.at[pl.ds(base, _COMB_ROWS)], i1_v)
        abufs = (a0_v, a1_v)
        ssem = (t0, t1)
        stores = [None, None]
        for c in range(_COMB_ITERS):
            ba = abufs[c & 1]
            sl = pl.ds(c * _COMB_CHUNK, _COMB_CHUNK)
            cp0 = pltpu.async_copy(ys_hbm.at[i0_v.at[sl]], ba, s0)
            cp1 = pltpu.async_copy(ys_hbm.at[i1_v.at[sl]], b_v, s1)
            cp0.wait()
            cp1.wait()

            def _add(k, _, ba=ba):
                r = k // (H // 16)
                col = (k % (H // 16)) * 16
                csl = pl.ds(col, 16)
                ba[r, csl] = ba[r, csl] + b_v[r, csl]
                return _

            lax.fori_loop(0, _VECS, _add, None)
            if stores[c & 1] is not None:
                stores[c & 1].wait()
            stores[c & 1] = pltpu.async_copy(
                ba, out_hbm.at[pl.ds(base + c * _COMB_CHUNK, _COMB_CHUNK)],
                ssem[c & 1])
        for st in stores:
            if st is not None:
                st.wait()

    return comb


def _combine(ys, pos0, pos1):
    return _build_combine()(ys, pos0, pos1)


def kernel(x, router_w, gate_w, up_w, down_w):
    flat = x.reshape(T, H)
    logits, i1, i2, w1, w2 = _router(flat, router_w)

    e_flat = jnp.concatenate([i1, i2], axis=1).reshape(P)
    w_flat = jnp.concatenate([w1, w2], axis=1).reshape(P)
    order = jnp.argsort(e_flat, stable=True)
    sorted_e = e_flat[order]
    counts = jnp.bincount(e_flat, length=E)
    group_start = jnp.concatenate(
        [jnp.zeros((1,), jnp.int32), jnp.cumsum(counts)[:-1].astype(jnp.int32)])
    padded = ((counts + R - 1) // R) * R
    padded_off = jnp.concatenate(
        [jnp.zeros((1,), jnp.int32), jnp.cumsum(padded)[:-1].astype(jnp.int32)])
    ranks = jnp.arange(P, dtype=jnp.int32) - group_start[sorted_e]
    dest = padded_off[sorted_e] + ranks
    tok = (order // K).astype(jnp.int32)
    tok_src = jnp.zeros((P_PAD,), jnp.int32).at[dest].set(tok)
    wrow = jnp.zeros((P_PAD,), jnp.float32).at[dest].set(w_flat[order])
    wrow = wrow.reshape(P_PAD, 1)
    posq = jnp.zeros((P,), jnp.int32).at[order].set(dest)
    pos0 = posq[0::2]
    pos1 = posq[1::2]
    nb_e = (padded // R).astype(jnp.int32)
    eid = jnp.repeat(jnp.arange(E, dtype=jnp.int32), nb_e,
                     total_repeat_length=NB)
    valid = (jnp.arange(NB) < jnp.sum(nb_e)).astype(jnp.int32)

    xs = _dispatch(flat, tok_src)
    ys = _gmm(eid, valid, xs, gate_w, up_w, down_w, wrow)
    out = _combine(ys, pos0, pos1)
    return out.reshape(1, T, H), logits


# speedup vs baseline: 1.2157x; 1.2157x over previous
import functools

import jax
import jax.numpy as jnp
from jax import lax
from jax.experimental import pallas as pl
from jax.experimental.pallas import tpu as pltpu
from jax.experimental.pallas import tpu_sc as plsc

T = 2048
H = 1024
F = 4096
E = 8
K = 2
P = T * K
R = 128
NB = P // R + E
P_PAD = NB * R
FT = 1024
NF = F // FT

NC = 2
NS = 16
NW = NC * NS


def _router_body(x_ref, w_ref, logits_ref, d1_ref, d2_ref, w1_ref, w2_ref,
                 cnt_ref):
    logits = jnp.dot(x_ref[...], w_ref[...], preferred_element_type=jnp.float32)
    logits_ref[...] = logits
    probs = jax.nn.softmax(logits, axis=-1)
    eidx = lax.broadcasted_iota(jnp.int32, (T, E), 1)
    m1 = jnp.max(probs, axis=1, keepdims=True)
    i1 = jnp.min(jnp.where(probs == m1, eidx, E), axis=1, keepdims=True)
    masked = jnp.where(eidx == i1, -1.0, probs)
    m2 = jnp.max(masked, axis=1, keepdims=True)
    i2 = jnp.min(jnp.where(masked == m2, eidx, E), axis=1, keepdims=True)
    s = m1 + m2
    w1_ref[...] = m1 / s
    w2_ref[...] = m2 / s

    c1 = (eidx == i1).astype(jnp.float32)
    c2 = (eidx == i2).astype(jnp.float32)
    c = c1 + c2
    tr = lax.broadcasted_iota(jnp.int32, (T, T), 0)
    tc = lax.broadcasted_iota(jnp.int32, (T, T), 1)
    tri = (tc < tr).astype(jnp.float32)
    run = jnp.dot(tri, c, preferred_element_type=jnp.float32)
    counts = jnp.sum(c, axis=0, keepdims=True)
    padded = jnp.ceil(counts * (1.0 / R)) * R
    er = lax.broadcasted_iota(jnp.int32, (E, E), 0)
    ec = lax.broadcasted_iota(jnp.int32, (E, E), 1)
    tri_e = (er < ec).astype(jnp.float32)
    poff = jnp.dot(padded, tri_e, preferred_element_type=jnp.float32)
    base = run + poff
    d1 = jnp.sum(c1 * base, axis=1, keepdims=True)
    d2 = jnp.sum(c2 * (base + c1), axis=1, keepdims=True)
    d1_ref[...] = d1.astype(jnp.int32)
    d2_ref[...] = d2.astype(jnp.int32)
    cnt_ref[...] = counts.astype(jnp.int32)


def _router(flat, router_w):
    return pl.pallas_call(
        _router_body,
        out_shape=(
            jax.ShapeDtypeStruct((T, E), jnp.float32),
            jax.ShapeDtypeStruct((T, 1), jnp.int32),
            jax.ShapeDtypeStruct((T, 1), jnp.int32),
            jax.ShapeDtypeStruct((T, 1), jnp.float32),
            jax.ShapeDtypeStruct((T, 1), jnp.float32),
            jax.ShapeDtypeStruct((1, E), jnp.int32),
        ),
    )(flat, router_w)


_DT = T // NW
_DC = 32
_DI = _DT // _DC


@functools.cache
def _sc_mesh():
    return plsc.VectorSubcoreMesh(core_axis_name="c", subcore_axis_name="s")


@functools.cache
def _build_dispatch():
    @functools.partial(
        pl.kernel,
        mesh=_sc_mesh(),
        out_type=jax.ShapeDtypeStruct((P_PAD, H), jnp.float32),
        scratch_types=[
            pltpu.VMEM((_DT,), jnp.int32),
            pltpu.VMEM((_DT,), jnp.int32),
            pltpu.VMEM((_DC, H), jnp.float32),
            pltpu.VMEM((_DC, H), jnp.float32),
            pltpu.SemaphoreType.DMA,
            pltpu.SemaphoreType.DMA,
            pltpu.SemaphoreType.DMA,
        ],
    )
    def disp(x_hbm, d1_hbm, d2_hbm, xs_hbm,
             i1_v, i2_v, xb0, xb1, g0, g1, s0):
        wid = lax.axis_index("s") * NC + lax.axis_index("c")
        base = wid * _DT
        pltpu.sync_copy(d1_hbm.at[pl.ds(base, _DT)], i1_v)
        pltpu.sync_copy(d2_hbm.at[pl.ds(base, _DT)], i2_v)
        xbufs = (xb0, xb1)
        gsem = (g0, g1)
        loads = [None, None]
        loads[0] = pltpu.async_copy(x_hbm.at[pl.ds(base, _DC)], xb0, g0)
        stores = []
        for c in range(_DI):
            if c + 1 < _DI:
                loads[(c + 1) & 1] = pltpu.async_copy(
                    x_hbm.at[pl.ds(base + (c + 1) * _DC, _DC)],
                    xbufs[(c + 1) & 1], gsem[(c + 1) & 1])
            loads[c & 1].wait()
            sl = pl.ds(c * _DC, _DC)
            stores.append(pltpu.async_copy(
                xbufs[c & 1], xs_hbm.at[i1_v.at[sl]], s0))
            stores.append(pltpu.async_copy(
                xbufs[c & 1], xs_hbm.at[i2_v.at[sl]], s0))
        for st in stores:
            st.wait()

    return disp


def _dispatch(flat, d1, d2):
    return _build_dispatch()(flat, d1, d2)


def _gmm_body(eid_ref, vld_ref, xs_ref, gw_ref, uw_ref, dw_ref, wr_ref, out_ref):
    j = pl.program_id(0)
    i = pl.program_id(1)

    @pl.when(vld_ref[i] == 1)
    def _():
        dims = (((1,), (0,)), ((), ()))
        xb = xs_ref[...]
        g = lax.dot_general(xb, gw_ref[0], dims,
                            preferred_element_type=jnp.float32)
        u = lax.dot_general(xb, uw_ref[0], dims,
                            preferred_element_type=jnp.float32)
        hmid = g / (1.0 + jnp.exp(-g)) * u
        partial = lax.dot_general(hmid, dw_ref[0], dims,
                                  preferred_element_type=jnp.float32)
        rows = pl.ds(i * R, R)

        @pl.when(j == 0)
        def _():
            out_ref[rows, :] = partial

        @pl.when(j > 0)
        def _():
            out_ref[rows, :] = out_ref[rows, :] + partial

        @pl.when(j == NF - 1)
        def _():
            out_ref[rows, :] = out_ref[rows, :] * wr_ref[...]


def _gmm(eid, valid, xs, gate_w, up_w, down_w, wrow):
    grid_spec = pltpu.PrefetchScalarGridSpec(
        num_scalar_prefetch=2,
        grid=(NF, NB),
        in_specs=[
            pl.BlockSpec((R, H), lambda j, i, eid, vld: (i, 0)),
            pl.BlockSpec((1, H, FT), lambda j, i, eid, vld: (eid[i], 0, j)),
            pl.BlockSpec((1, H, FT), lambda j, i, eid, vld: (eid[i], 0, j)),
            pl.BlockSpec((1, FT, H), lambda j, i, eid, vld: (eid[i], j, 0)),
            pl.BlockSpec((R, 1), lambda j, i, eid, vld: (i, 0)),
        ],
        out_specs=pl.BlockSpec((P_PAD, H), lambda j, i, eid, vld: (0, 0)),
    )
    return pl.pallas_call(
        _gmm_body,
        grid_spec=grid_spec,
        out_shape=jax.ShapeDtypeStruct((P_PAD, H), jnp.float32),
        compiler_params=pltpu.CompilerParams(
            dimension_semantics=("arbitrary", "arbitrary"),
        ),
    )(eid, valid, xs, gate_w, up_w, down_w, wrow)


_COMB_ROWS = T // NW
_COMB_CHUNK = 32
_COMB_ITERS = _COMB_ROWS // _COMB_CHUNK
_VECS = _COMB_CHUNK * H // 16


@functools.cache
def _build_combine():
    @functools.partial(
        pl.kernel,
        mesh=_sc_mesh(),
        out_type=jax.ShapeDtypeStruct((T, H), jnp.float32),
        scratch_types=[
            pltpu.VMEM((_COMB_ROWS,), jnp.int32),
            pltpu.VMEM((_COMB_ROWS,), jnp.int32),
            pltpu.VMEM((_COMB_CHUNK, H), jnp.float32),
            pltpu.VMEM((_COMB_CHUNK, H), jnp.float32),
            pltpu.VMEM((_COMB_CHUNK, H), jnp.float32),
            pltpu.SemaphoreType.DMA,
            pltpu.SemaphoreType.DMA,
            pltpu.SemaphoreType.DMA,
            pltpu.SemaphoreType.DMA,
        ],
    )
    def comb(ys_hbm, p0_hbm, p1_hbm, out_hbm,
             i0_v, i1_v, a0_v, a1_v, b_v, s0, s1, t0, t1):
        wid = lax.axis_index("s") * NC + lax.axis_index("c")
        base = wid * _COMB_ROWS
        pltpu.sync_copy(p0_hbm.at[pl.ds(base, _COMB_ROWS)], i0_v)
        pltpu.sync_copy(p1_hbm.at[pl.ds(base, _COMB_ROWS)], i1_v)
        abufs = (a0_v, a1_v)
        ssem = (t0, t1)
        stores = [None, None]
        for c in range(_COMB_ITERS):
            ba = abufs[c & 1]
            sl = pl.ds(c * _COMB_CHUNK, _COMB_CHUNK)
            cp0 = pltpu.async_copy(ys_hbm.at[i0_v.at[sl]], ba, s0)
            cp1 = pltpu.async_copy(ys_hbm.at[i1_v.at[sl]], b_v, s1)
            cp0.wait()
            cp1.wait()

            def _add(k, _, ba=ba):
                r = k // (H // 16)
                col = (k % (H // 16)) * 16
                csl = pl.ds(col, 16)
                ba[r, csl] = ba[r, csl] + b_v[r, csl]
                return _

            lax.fori_loop(0, _VECS, _add, None)
            if stores[c & 1] is not None:
                stores[c & 1].wait()
            stores[c & 1] = pltpu.async_copy(
                ba, out_hbm.at[pl.ds(base + c * _COMB_CHUNK, _COMB_CHUNK)],
                ssem[c & 1])
        for st in stores:
            if st is not None:
                st.wait()

    return comb


def _combine(ys, pos0, pos1):
    return _build_combine()(ys, pos0, pos1)


def kernel(x, router_w, gate_w, up_w, down_w):
    flat = x.reshape(T, H)
    logits, d1, d2, w1, w2, cnt = _router(flat, router_w)

    counts = cnt.reshape(E)
    nb_e = (counts + R - 1) // R
    eid = jnp.repeat(jnp.arange(E, dtype=jnp.int32), nb_e,
                     total_repeat_length=NB)
    valid = (jnp.arange(NB, dtype=jnp.int32) < jnp.sum(nb_e)).astype(jnp.int32)
    d1f = d1.reshape(T)
    d2f = d2.reshape(T)
    wrow = (jnp.zeros((P_PAD,), jnp.float32)
            .at[d1f].set(w1.reshape(T))
            .at[d2f].set(w2.reshape(T))
            .reshape(P_PAD, 1))

    xs = _dispatch(flat, d1f, d2f)
    ys = _gmm(eid, valid, xs, gate_w, up_w, down_w, wrow)
    out = _combine(ys, d1f, d2f)
    return out.reshape(1, T, H), logits


# speedup vs baseline: 1.3297x; 1.0938x over previous
import functools

import jax
import jax.numpy as jnp
from jax import lax
from jax.experimental import pallas as pl
from jax.experimental.pallas import tpu as pltpu
from jax.experimental.pallas import tpu_sc as plsc

T = 2048
H = 1024
F = 4096
E = 8
K = 2
P = T * K
R = 256
NB = P // R + E
P_PAD = NB * R
FT = 1024
NF = F // FT

NC = 2
NS = 16
NW = NC * NS


def _router_body(x_ref, w_ref, logits_ref, d1_ref, d2_ref, w1_ref, w2_ref,
                 cnt_ref):
    logits = jnp.dot(x_ref[...], w_ref[...], preferred_element_type=jnp.float32)
    logits_ref[...] = logits
    probs = jax.nn.softmax(logits, axis=-1)
    eidx = lax.broadcasted_iota(jnp.int32, (T, E), 1)
    m1 = jnp.max(probs, axis=1, keepdims=True)
    i1 = jnp.min(jnp.where(probs == m1, eidx, E), axis=1, keepdims=True)
    masked = jnp.where(eidx == i1, -1.0, probs)
    m2 = jnp.max(masked, axis=1, keepdims=True)
    i2 = jnp.min(jnp.where(masked == m2, eidx, E), axis=1, keepdims=True)
    s = m1 + m2
    w1_ref[...] = m1 / s
    w2_ref[...] = m2 / s

    c1 = (eidx == i1).astype(jnp.float32)
    c2 = (eidx == i2).astype(jnp.float32)
    c = c1 + c2
    tr = lax.broadcasted_iota(jnp.int32, (T, T), 0)
    tc = lax.broadcasted_iota(jnp.int32, (T, T), 1)
    tri = (tc < tr).astype(jnp.float32)
    run = jnp.dot(tri, c, preferred_element_type=jnp.float32)
    counts = jnp.sum(c, axis=0, keepdims=True)
    padded = jnp.ceil(counts * (1.0 / R)) * R
    er = lax.broadcasted_iota(jnp.int32, (E, E), 0)
    ec = lax.broadcasted_iota(jnp.int32, (E, E), 1)
    tri_e = (er < ec).astype(jnp.float32)
    poff = jnp.dot(padded, tri_e, preferred_element_type=jnp.float32)
    base = run + poff
    d1 = jnp.sum(c1 * base, axis=1, keepdims=True)
    d2 = jnp.sum(c2 * (base + c1), axis=1, keepdims=True)
    d1_ref[...] = d1.astype(jnp.int32)
    d2_ref[...] = d2.astype(jnp.int32)
    cnt_ref[...] = counts.astype(jnp.int32)


def _router(flat, router_w):
    return pl.pallas_call(
        _router_body,
        out_shape=(
            jax.ShapeDtypeStruct((T, E), jnp.float32),
            jax.ShapeDtypeStruct((T, 1), jnp.int32),
            jax.ShapeDtypeStruct((T, 1), jnp.int32),
            jax.ShapeDtypeStruct((T, 1), jnp.float32),
            jax.ShapeDtypeStruct((T, 1), jnp.float32),
            jax.ShapeDtypeStruct((1, E), jnp.int32),
        ),
    )(flat, router_w)


_DT = T // NW
_DC = 32
_DI = _DT // _DC


@functools.cache
def _sc_mesh():
    return plsc.VectorSubcoreMesh(core_axis_name="c", subcore_axis_name="s")


@functools.cache
def _build_dispatch():
    @functools.partial(
        pl.kernel,
        mesh=_sc_mesh(),
        out_type=jax.ShapeDtypeStruct((P_PAD, H), jnp.float32),
        scratch_types=[
            pltpu.VMEM((_DT,), jnp.int32),
            pltpu.VMEM((_DT,), jnp.int32),
            pltpu.VMEM((_DC, H), jnp.float32),
            pltpu.VMEM((_DC, H), jnp.float32),
            pltpu.SemaphoreType.DMA,
            pltpu.SemaphoreType.DMA,
            pltpu.SemaphoreType.DMA,
        ],
    )
    def disp(x_hbm, d1_hbm, d2_hbm, xs_hbm,
             i1_v, i2_v, xb0, xb1, g0, g1, s0):
        wid = lax.axis_index("s") * NC + lax.axis_index("c")
        base = wid * _DT
        pltpu.sync_copy(d1_hbm.at[pl.ds(base, _DT)], i1_v)
        pltpu.sync_copy(d2_hbm.at[pl.ds(base, _DT)], i2_v)
        xbufs = (xb0, xb1)
        gsem = (g0, g1)
        loads = [None, None]
        loads[0] = pltpu.async_copy(x_hbm.at[pl.ds(base, _DC)], xb0, g0)
        stores = []
        for c in range(_DI):
            if c + 1 < _DI:
                loads[(c + 1) & 1] = pltpu.async_copy(
                    x_hbm.at[pl.ds(base + (c + 1) * _DC, _DC)],
                    xbufs[(c + 1) & 1], gsem[(c + 1) & 1])
            loads[c & 1].wait()
            sl = pl.ds(c * _DC, _DC)
            stores.append(pltpu.async_copy(
                xbufs[c & 1], xs_hbm.at[i1_v.at[sl]], s0))
            stores.append(pltpu.async_copy(
                xbufs[c & 1], xs_hbm.at[i2_v.at[sl]], s0))
        for st in stores:
            st.wait()

    return disp


def _dispatch(flat, d1, d2):
    return _build_dispatch()(flat, d1, d2)


def _gmm_body(eid_ref, vld_ref, xs_ref, gw_ref, uw_ref, dw_ref, wr_ref, out_ref):
    j = pl.program_id(0)
    i = pl.program_id(1)

    @pl.when(vld_ref[i] == 1)
    def _():
        dims = (((1,), (0,)), ((), ()))
        xb = xs_ref[...]
        g = lax.dot_general(xb, gw_ref[0], dims,
                            preferred_element_type=jnp.float32)
        u = lax.dot_general(xb, uw_ref[0], dims,
                            preferred_element_type=jnp.float32)
        hmid = g / (1.0 + jnp.exp(-g)) * u
        partial = lax.dot_general(hmid, dw_ref[0], dims,
                                  preferred_element_type=jnp.float32)
        rows = pl.ds(i * R, R)

        @pl.when(j == 0)
        def _():
            out_ref[rows, :] = partial

        @pl.when(j > 0)
        def _():
            out_ref[rows, :] = out_ref[rows, :] + partial

        @pl.when(j == NF - 1)
        def _():
            out_ref[rows, :] = out_ref[rows, :] * wr_ref[...]


def _gmm(eid, valid, xs, gate_w, up_w, down_w, wrow):
    grid_spec = pltpu.PrefetchScalarGridSpec(
        num_scalar_prefetch=2,
        grid=(NF, NB),
        in_specs=[
            pl.BlockSpec((R, H), lambda j, i, eid, vld: (i, 0)),
            pl.BlockSpec((1, H, FT), lambda j, i, eid, vld: (eid[i], 0, j)),
            pl.BlockSpec((1, H, FT), lambda j, i, eid, vld: (eid[i], 0, j)),
            pl.BlockSpec((1, FT, H), lambda j, i, eid, vld: (eid[i], j, 0)),
            pl.BlockSpec((R, 1), lambda j, i, eid, vld: (i, 0)),
        ],
        out_specs=pl.BlockSpec((P_PAD, H), lambda j, i, eid, vld: (0, 0)),
    )
    return pl.pallas_call(
        _gmm_body,
        grid_spec=grid_spec,
        out_shape=jax.ShapeDtypeStruct((P_PAD, H), jnp.float32),
        compiler_params=pltpu.CompilerParams(
            dimension_semantics=("arbitrary", "arbitrary"),
        ),
    )(eid, valid, xs, gate_w, up_w, down_w, wrow)


_COMB_ROWS = T // NW
_COMB_CHUNK = 32
_COMB_ITERS = _COMB_ROWS // _COMB_CHUNK
_VECS = _COMB_CHUNK * H // 16


@functools.cache
def _build_combine():
    @functools.partial(
        pl.kernel,
        mesh=_sc_mesh(),
        out_type=jax.ShapeDtypeStruct((T, H), jnp.float32),
        scratch_types=[
            pltpu.VMEM((_COMB_ROWS,), jnp.int32),
            pltpu.VMEM((_COMB_ROWS,), jnp.int32),
            pltpu.VMEM((_COMB_CHUNK, H), jnp.float32),
            pltpu.VMEM((_COMB_CHUNK, H), jnp.float32),
            pltpu.VMEM((_COMB_CHUNK, H), jnp.float32),
            pltpu.SemaphoreType.DMA,
            pltpu.SemaphoreType.DMA,
            pltpu.SemaphoreType.DMA,
            pltpu.SemaphoreType.DMA,
        ],
    )
    def comb(ys_hbm, p0_hbm, p1_hbm, out_hbm,
             i0_v, i1_v, a0_v, a1_v, b_v, s0, s1, t0, t1):
        wid = lax.axis_index("s") * NC + lax.axis_index("c")
        base = wid * _COMB_ROWS
        pltpu.sync_copy(p0_hbm.at[pl.ds(base, _COMB_ROWS)], i0_v)
        pltpu.sync_copy(p1_hbm.at[pl.ds(base, _COMB_ROWS)], i1_v)
        abufs = (a0_v, a1_v)
        ssem = (t0, t1)
        stores = [None, None]
        for c in range(_COMB_ITERS):
            ba = abufs[c & 1]
            sl = pl.ds(c * _COMB_CHUNK, _COMB_CHUNK)
            cp0 = pltpu.async_copy(ys_hbm.at[i0_v.at[sl]], ba, s0)
            cp1 = pltpu.async_copy(ys_hbm.at[i1_v.at[sl]], b_v, s1)
            cp0.wait()
            cp1.wait()

            def _add(k, _, ba=ba):
                r = k // (H // 16)
                col = (k % (H // 16)) * 16
                csl = pl.ds(col, 16)
                ba[r, csl] = ba[r, csl] + b_v[r, csl]
                return _

            lax.fori_loop(0, _VECS, _add, None)
            if stores[c & 1] is not None:
                stores[c & 1].wait()
            stores[c & 1] = pltpu.async_copy(
                ba, out_hbm.at[pl.ds(base + c * _COMB_CHUNK, _COMB_CHUNK)],
                ssem[c & 1])
        for st in stores:
            if st is not None:
                st.wait()

    return comb


def _combine(ys, pos0, pos1):
    return _build_combine()(ys, pos0, pos1)


def kernel(x, router_w, gate_w, up_w, down_w):
    flat = x.reshape(T, H)
    logits, d1, d2, w1, w2, cnt = _router(flat, router_w)

    counts = cnt.reshape(E)
    nb_e = (counts + R - 1) // R
    eid = jnp.repeat(jnp.arange(E, dtype=jnp.int32), nb_e,
                     total_repeat_length=NB)
    valid = (jnp.arange(NB, dtype=jnp.int32) < jnp.sum(nb_e)).astype(jnp.int32)
    d1f = d1.reshape(T)
    d2f = d2.reshape(T)
    wrow = (jnp.zeros((P_PAD,), jnp.float32)
            .at[d1f].set(w1.reshape(T))
            .at[d2f].set(w2.reshape(T))
            .reshape(P_PAD, 1))

    xs = _dispatch(flat, d1f, d2f)
    ys = _gmm(eid, valid, xs, gate_w, up_w, down_w, wrow)
    out = _combine(ys, d1f, d2f)
    return out.reshape(1, T, H), logits


# speedup vs baseline: 1.3801x; 1.0379x over previous
import functools

import jax
import jax.numpy as jnp
from jax import lax
from jax.experimental import pallas as pl
from jax.experimental.pallas import tpu as pltpu
from jax.experimental.pallas import tpu_sc as plsc

T = 2048
H = 1024
F = 4096
E = 8
K = 2
P = T * K
R = 256
NB = P // R + E
P_PAD = NB * R
FT = 1024
NF = F // FT

NC = 2
NS = 16
NW = NC * NS


def _router_body(x_ref, w_ref, logits_ref, d1_ref, d2_ref, w1_ref, w2_ref,
                 cnt_ref):
    logits = jnp.dot(x_ref[...], w_ref[...], preferred_element_type=jnp.float32)
    logits_ref[...] = logits
    probs = jax.nn.softmax(logits, axis=-1)
    eidx = lax.broadcasted_iota(jnp.int32, (T, E), 1)
    m1 = jnp.max(probs, axis=1, keepdims=True)
    i1 = jnp.min(jnp.where(probs == m1, eidx, E), axis=1, keepdims=True)
    masked = jnp.where(eidx == i1, -1.0, probs)
    m2 = jnp.max(masked, axis=1, keepdims=True)
    i2 = jnp.min(jnp.where(masked == m2, eidx, E), axis=1, keepdims=True)
    s = m1 + m2
    w1_ref[...] = m1 / s
    w2_ref[...] = m2 / s

    c1 = (eidx == i1).astype(jnp.float32)
    c2 = (eidx == i2).astype(jnp.float32)
    c = c1 + c2
    tr = lax.broadcasted_iota(jnp.int32, (T, T), 0)
    tc = lax.broadcasted_iota(jnp.int32, (T, T), 1)
    tri = (tc < tr).astype(jnp.float32)
    run = jnp.dot(tri, c, preferred_element_type=jnp.float32)
    counts = jnp.sum(c, axis=0, keepdims=True)
    padded = jnp.ceil(counts * (1.0 / R)) * R
    er = lax.broadcasted_iota(jnp.int32, (E, E), 0)
    ec = lax.broadcasted_iota(jnp.int32, (E, E), 1)
    tri_e = (er < ec).astype(jnp.float32)
    poff = jnp.dot(padded, tri_e, preferred_element_type=jnp.float32)
    base = run + poff
    d1 = jnp.sum(c1 * base, axis=1, keepdims=True)
    d2 = jnp.sum(c2 * (base + c1), axis=1, keepdims=True)
    d1_ref[...] = d1.astype(jnp.int32)
    d2_ref[...] = d2.astype(jnp.int32)
    cnt_ref[...] = counts.astype(jnp.int32)


def _router(flat, router_w):
    return pl.pallas_call(
        _router_body,
        out_shape=(
            jax.ShapeDtypeStruct((T, E), jnp.float32),
            jax.ShapeDtypeStruct((T, 1), jnp.int32),
            jax.ShapeDtypeStruct((T, 1), jnp.int32),
            jax.ShapeDtypeStruct((T, 1), jnp.float32),
            jax.ShapeDtypeStruct((T, 1), jnp.float32),
            jax.ShapeDtypeStruct((1, E), jnp.int32),
        ),
    )(flat, router_w)


_DT = T // NW
_DC = 32
_DI = _DT // _DC


@functools.cache
def _sc_mesh():
    return plsc.VectorSubcoreMesh(core_axis_name="c", subcore_axis_name="s")


@functools.cache
def _build_dispatch():
    @functools.partial(
        pl.kernel,
        mesh=_sc_mesh(),
        out_type=jax.ShapeDtypeStruct((P_PAD, H), jnp.float32),
        scratch_types=[
            pltpu.VMEM((_DT,), jnp.int32),
            pltpu.VMEM((_DT,), jnp.int32),
            pltpu.VMEM((_DC, H), jnp.float32),
            pltpu.VMEM((_DC, H), jnp.float32),
            pltpu.SemaphoreType.DMA,
            pltpu.SemaphoreType.DMA,
            pltpu.SemaphoreType.DMA,
        ],
    )
    def disp(x_hbm, d1_hbm, d2_hbm, xs_hbm,
             i1_v, i2_v, xb0, xb1, g0, g1, s0):
        wid = lax.axis_index("s") * NC + lax.axis_index("c")
        base = wid * _DT
        pltpu.sync_copy(d1_hbm.at[pl.ds(base, _DT)], i1_v)
        pltpu.sync_copy(d2_hbm.at[pl.ds(base, _DT)], i2_v)
        xbufs = (xb0, xb1)
        gsem = (g0, g1)
        loads = [None, None]
        loads[0] = pltpu.async_copy(x_hbm.at[pl.ds(base, _DC)], xb0, g0)
        stores = []
        for c in range(_DI):
            if c + 1 < _DI:
                loads[(c + 1) & 1] = pltpu.async_copy(
                    x_hbm.at[pl.ds(base + (c + 1) * _DC, _DC)],
                    xbufs[(c + 1) & 1], gsem[(c + 1) & 1])
            loads[c & 1].wait()
            sl = pl.ds(c * _DC, _DC)
            stores.append(pltpu.async_copy(
                xbufs[c & 1], xs_hbm.at[i1_v.at[sl]], s0))
            stores.append(pltpu.async_copy(
                xbufs[c & 1], xs_hbm.at[i2_v.at[sl]], s0))
        for st in stores:
            st.wait()

    return disp


def _dispatch(flat, d1, d2):
    return _build_dispatch()(flat, d1, d2)


def _gmm_body(eid_ref, vld_ref, xs_ref, gw_ref, uw_ref, dw_ref, wr_ref, out_ref):
    j = pl.program_id(0)
    i = pl.program_id(1)

    @pl.when(vld_ref[i] == 1)
    def _():
        dims = (((1,), (0,)), ((), ()))
        xb = xs_ref[...]
        g = lax.dot_general(xb, gw_ref[0], dims,
                            preferred_element_type=jnp.float32)
        u = lax.dot_general(xb, uw_ref[0], dims,
                            preferred_element_type=jnp.float32)
        hmid = g / (1.0 + jnp.exp(-g)) * u
        partial = lax.dot_general(hmid, dw_ref[0], dims,
                                  preferred_element_type=jnp.float32)
        rows = pl.ds(i * R, R)

        @pl.when(j == 0)
        def _():
            out_ref[rows, :] = partial

        @pl.when(j > 0)
        def _():
            out_ref[rows, :] = out_ref[rows, :] + partial

        @pl.when(j == NF - 1)
        def _():
            out_ref[rows, :] = out_ref[rows, :] * wr_ref[...]


def _gmm(eid, valid, xs, gate_w, up_w, down_w, wrow):
    grid_spec = pltpu.PrefetchScalarGridSpec(
        num_scalar_prefetch=2,
        grid=(NF, NB),
        in_specs=[
            pl.BlockSpec((R, H), lambda j, i, eid, vld: (i, 0)),
            pl.BlockSpec((1, H, FT), lambda j, i, eid, vld: (eid[i], 0, j)),
            pl.BlockSpec((1, H, FT), lambda j, i, eid, vld: (eid[i], 0, j)),
            pl.BlockSpec((1, FT, H), lambda j, i, eid, vld: (eid[i], j, 0)),
            pl.BlockSpec((R, 1), lambda j, i, eid, vld: (i, 0)),
        ],
        out_specs=pl.BlockSpec((P_PAD, H), lambda j, i, eid, vld: (0, 0)),
    )
    return pl.pallas_call(
        _gmm_body,
        grid_spec=grid_spec,
        out_shape=jax.ShapeDtypeStruct((P_PAD, H), jnp.float32),
        compiler_params=pltpu.CompilerParams(
            dimension_semantics=("arbitrary", "arbitrary"),
        ),
    )(eid, valid, xs, gate_w, up_w, down_w, wrow)


_COMB_ROWS = T // NW
_COMB_CHUNK = 32
_COMB_ITERS = _COMB_ROWS // _COMB_CHUNK
_VECS = _COMB_CHUNK * H // 16


@functools.cache
def _build_combine():
    @functools.partial(
        pl.kernel,
        mesh=_sc_mesh(),
        out_type=jax.ShapeDtypeStruct((T, H), jnp.float32),
        scratch_types=[
            pltpu.VMEM((_COMB_ROWS,), jnp.int32),
            pltpu.VMEM((_COMB_ROWS,), jnp.int32),
            pltpu.VMEM((_COMB_CHUNK, H), jnp.float32),
            pltpu.VMEM((_COMB_CHUNK, H), jnp.float32),
            pltpu.VMEM((_COMB_CHUNK, H), jnp.float32),
            pltpu.SemaphoreType.DMA,
            pltpu.SemaphoreType.DMA,
            pltpu.SemaphoreType.DMA,
            pltpu.SemaphoreType.DMA,
        ],
    )
    def comb(ys_hbm, p0_hbm, p1_hbm, out_hbm,
             i0_v, i1_v, a0_v, a1_v, b_v, s0, s1, t0, t1):
        wid = lax.axis_index("s") * NC + lax.axis_index("c")
        base = wid * _COMB_ROWS
        pltpu.sync_copy(p0_hbm.at[pl.ds(base, _COMB_ROWS)], i0_v)
        pltpu.sync_copy(p1_hbm.at[pl.ds(base, _COMB_ROWS)], i1_v)
        abufs = (a0_v, a1_v)
        ssem = (t0, t1)
        stores = [None, None]
        for c in range(_COMB_ITERS):
            ba = abufs[c & 1]
            sl = pl.ds(c * _COMB_CHUNK, _COMB_CHUNK)
            cp0 = pltpu.async_copy(ys_hbm.at[i0_v.at[sl]], ba, s0)
            cp1 = pltpu.async_copy(ys_hbm.at[i1_v.at[sl]], b_v, s1)
            cp0.wait()
            cp1.wait()

            def _add(k, _, ba=ba):
                r = k // (H // 128)
                cbase = (k % (H // 128)) * 128
                for u in range(8):
                    csl = pl.ds(cbase + u * 16, 16)
                    ba[r, csl] = ba[r, csl] + b_v[r, csl]
                return _

            lax.fori_loop(0, _VECS // 8, _add, None)
            if stores[c & 1] is not None:
                stores[c & 1].wait()
            stores[c & 1] = pltpu.async_copy(
                ba, out_hbm.at[pl.ds(base + c * _COMB_CHUNK, _COMB_CHUNK)],
                ssem[c & 1])
        for st in stores:
            if st is not None:
                st.wait()

    return comb


def _combine(ys, pos0, pos1):
    return _build_combine()(ys, pos0, pos1)


def kernel(x, router_w, gate_w, up_w, down_w):
    flat = x.reshape(T, H)
    logits, d1, d2, w1, w2, cnt = _router(flat, router_w)

    counts = cnt.reshape(E)
    nb_e = (counts + R - 1) // R
    eid = jnp.repeat(jnp.arange(E, dtype=jnp.int32), nb_e,
                     total_repeat_length=NB)
    valid = (jnp.arange(NB, dtype=jnp.int32) < jnp.sum(nb_e)).astype(jnp.int32)
    d1f = d1.reshape(T)
    d2f = d2.reshape(T)
    wrow = (jnp.zeros((P_PAD,), jnp.float32)
            .at[d1f].set(w1.reshape(T))
            .at[d2f].set(w2.reshape(T))
            .reshape(P_PAD, 1))

    xs = _dispatch(flat, d1f, d2f)
    ys = _gmm(eid, valid, xs, gate_w, up_w, down_w, wrow)
    out = _combine(ys, d1f, d2f)
    return out.reshape(1, T, H), logits
